# baseline (speedup 1.0000x reference)
import functools

import jax
import jax.numpy as jnp
import numpy as np
from jax import lax
from jax.experimental import pallas as pl
from jax.experimental.pallas import tpu as pltpu
from jax.experimental.pallas import tpu_sc as plsc

_NUM_CORES = 2
_NUM_SUBCORES = 16
_LANES = 16
_NW = _NUM_CORES * _NUM_SUBCORES

_NBUF = 8
_AHEAD = 6
_BLK = 2


def _make_emb_kernel(batch, seq_len, dim):
    bpw = batch // _NW
    s_hi = seq_len // 8
    n_groups = seq_len // _NBUF
    cgroups = dim // _LANES
    mesh = plsc.VectorSubcoreMesh(
        core_axis_name="c", subcore_axis_name="s",
        num_cores=_NUM_CORES, num_subcores=_NUM_SUBCORES)

    @functools.partial(
        pl.kernel,
        out_type=jax.ShapeDtypeStruct(
            (seq_len, dim // 8, _NW, 8, 128), jnp.float32),
        mesh=mesh,
        scratch_types=[
            pltpu.VMEM((s_hi, 8, bpw), jnp.int32),
            pltpu.VMEM((_NBUF, bpw, dim), jnp.float32),
            pltpu.VMEM((_BLK, dim, bpw + 1), jnp.float32),
            pltpu.VMEM((seq_len, dim), jnp.float32),
            pltpu.SemaphoreType.DMA((_NBUF,)),
            pltpu.SemaphoreType.DMA((_BLK,)),
        ],
        compiler_params=pltpu.CompilerParams(
            use_tc_tiling_on_sc=False, needs_layout_passes=False),
    )
    def emb(ids_hbm, table_hbm, pe_hbm, out_hbm, idx_v, rows, block, pe_v,
            gsem, ssem):
        wid = lax.axis_index("c") * _NUM_SUBCORES + lax.axis_index("s")
        pltpu.sync_copy(pe_hbm, pe_v)
        pltpu.sync_copy(ids_hbm.at[wid], idx_v)
        iota = lax.iota(jnp.int32, _LANES)
        cvecs = [g * _LANES + iota for g in range(cgroups)]

        def gather_start(s, slot):
            pltpu.async_copy(
                table_hbm.at[idx_v.at[s // 8, lax.rem(s, 8)]],
                rows.at[slot], gsem.at[slot])

        def gather_wait(s, slot):
            pltpu.make_async_copy(
                table_hbm.at[idx_v.at[s // 8, lax.rem(s, 8)]],
                rows.at[slot], gsem.at[slot]).wait()

        def store_start(s, p):
            for chi in range(dim // 8):
                pltpu.async_copy(block.at[p, pl.ds(chi * 8, 8), pl.ds(0, 128)],
                                 out_hbm.at[s, chi, wid], ssem.at[p])

        def store_wait(s, p):
            for chi in range(dim // 8):
                pltpu.make_async_copy(
                    block.at[p, pl.ds(chi * 8, 8), pl.ds(0, 128)],
                    out_hbm.at[s, chi, wid], ssem.at[p]).wait()

        for b in range(_AHEAD):
            gather_start(b, b)

        def group_body(g, carry):
            for b in range(_NBUF):
                s = g * _NBUF + b
                p = b % _BLK
                gather_wait(s, b)

                @pl.when(s >= _BLK)
                def _drain():
                    store_wait(s - _BLK, p)

                pe_gs = [pe_v[s, pl.ds(gg * _LANES, _LANES)]
                         for gg in range(cgroups)]

                @plsc.parallel_loop(0, bpw, 1, unroll=8)
                def blo_body(blo):
                    blo_b = lax.broadcast(blo, (_LANES,))
                    for gg in range(cgroups):
                        val = rows[b, blo, pl.ds(gg * _LANES, _LANES)]
                        plsc.store_scatter(block.at[p], [cvecs[gg], blo_b],
                                           val + pe_gs[gg])

                f = s + _AHEAD

                @pl.when(f < seq_len)
                def _fire():
                    gather_start(f, (b + _AHEAD) % _NBUF)

                store_start(s, p)
            return carry

        lax.fori_loop(0, n_groups, group_body, 0)
        store_wait(seq_len - 2, 0)
        store_wait(seq_len - 1, 1)

    return emb


def _pos_encoding(seq_len, dim):
    pos = jnp.arange(seq_len, dtype=jnp.float32)[:, None]
    div = jnp.exp(
        jnp.arange(0, dim, 2, dtype=jnp.float32) * (-np.log(10000.0) / dim))
    pe = jnp.zeros((seq_len, dim), dtype=jnp.float32)
    pe = pe.at[:, 0::2].set(jnp.sin(pos * div))
    pe = pe.at[:, 1::2].set(jnp.cos(pos * div))
    return pe


def kernel(input_ids, table):
    batch, seq_len = input_ids.shape
    _, dim = table.shape
    assert batch == _NW * 128 and seq_len % 8 == 0
    assert dim % _LANES == 0 and dim % 8 == 0
    s_hi = seq_len // 8
    ids4 = (input_ids.astype(jnp.int32).T
            .reshape(s_hi, 8, _NW, 128).transpose(2, 0, 1, 3))
    pe = _pos_encoding(seq_len, dim)
    out5 = _make_emb_kernel(batch, seq_len, dim)(ids4, table, pe)
    return out5.transpose(2, 4, 0, 1, 3).reshape(batch, seq_len, dim)


# speedup vs baseline: 1.0322x; 1.0322x over previous
import functools

import jax
import jax.numpy as jnp
import numpy as np
from jax import lax
from jax.experimental import pallas as pl
from jax.experimental.pallas import tpu as pltpu
from jax.experimental.pallas import tpu_sc as plsc

_NUM_CORES = 2
_NUM_SUBCORES = 16
_LANES = 16
_NW = _NUM_CORES * _NUM_SUBCORES

_NBUF = 4
_AHEAD = 3
_BLK = 4


def _make_emb_kernel(batch, seq_len, dim):
    bpw = batch // _NW
    s_hi = seq_len // 8
    n_groups = seq_len // _NBUF
    cgroups = dim // _LANES
    mesh = plsc.VectorSubcoreMesh(
        core_axis_name="c", subcore_axis_name="s",
        num_cores=_NUM_CORES, num_subcores=_NUM_SUBCORES)

    @functools.partial(
        pl.kernel,
        out_type=jax.ShapeDtypeStruct(
            (seq_len, dim // 8, _NW, 8, 128), jnp.float32),
        mesh=mesh,
        scratch_types=[
            pltpu.VMEM((s_hi, 8, bpw), jnp.int32),
            pltpu.VMEM((_NBUF, bpw, dim), jnp.float32),
            pltpu.VMEM((_BLK, dim, bpw + 1), jnp.float32),
            pltpu.VMEM((seq_len, dim), jnp.float32),
            pltpu.SemaphoreType.DMA((_NBUF,)),
            pltpu.SemaphoreType.DMA((_BLK,)),
        ],
        compiler_params=pltpu.CompilerParams(
            use_tc_tiling_on_sc=False, needs_layout_passes=False),
    )
    def emb(ids_hbm, table_hbm, pe_hbm, out_hbm, idx_v, rows, block, pe_v,
            gsem, ssem):
        wid = lax.axis_index("c") * _NUM_SUBCORES + lax.axis_index("s")
        pltpu.sync_copy(pe_hbm, pe_v)
        pltpu.sync_copy(ids_hbm.at[wid], idx_v)
        iota = lax.iota(jnp.int32, _LANES)
        cvecs = [g * _LANES + iota for g in range(cgroups)]

        def gather_start(s, slot):
            pltpu.async_copy(
                table_hbm.at[idx_v.at[s // 8, lax.rem(s, 8)]],
                rows.at[slot], gsem.at[slot])

        def gather_wait(s, slot):
            pltpu.make_async_copy(
                table_hbm.at[idx_v.at[s // 8, lax.rem(s, 8)]],
                rows.at[slot], gsem.at[slot]).wait()

        def store_start(s, p):
            for chi in range(dim // 8):
                pltpu.async_copy(block.at[p, pl.ds(chi * 8, 8), pl.ds(0, 128)],
                                 out_hbm.at[s, chi, wid], ssem.at[p])

        def store_wait(s, p):
            for chi in range(dim // 8):
                pltpu.make_async_copy(
                    block.at[p, pl.ds(chi * 8, 8), pl.ds(0, 128)],
                    out_hbm.at[s, chi, wid], ssem.at[p]).wait()

        for b in range(_AHEAD):
            gather_start(b, b)

        def group_body(g, carry):
            for b in range(_NBUF):
                s = g * _NBUF + b
                p = b % _BLK
                gather_wait(s, b)

                @pl.when(s >= _BLK)
                def _drain():
                    store_wait(s - _BLK, p)

                pe_gs = [pe_v[s, pl.ds(gg * _LANES, _LANES)]
                         for gg in range(cgroups)]

                @plsc.parallel_loop(0, bpw, 1, unroll=8)
                def blo_body(blo):
                    blo_b = lax.broadcast(blo, (_LANES,))
                    for gg in range(cgroups):
                        val = rows[b, blo, pl.ds(gg * _LANES, _LANES)]
                        plsc.store_scatter(block.at[p], [cvecs[gg], blo_b],
                                           val + pe_gs[gg])

                f = s + _AHEAD

                @pl.when(f < seq_len)
                def _fire():
                    gather_start(f, (b + _AHEAD) % _NBUF)

                store_start(s, p)
            return carry

        lax.fori_loop(0, n_groups, group_body, 0)
        for i in range(_BLK):
            s_tail = seq_len - _BLK + i
            store_wait(s_tail, s_tail % _BLK)

    return emb


def _pos_encoding(seq_len, dim):
    pos = jnp.arange(seq_len, dtype=jnp.float32)[:, None]
    div = jnp.exp(
        jnp.arange(0, dim, 2, dtype=jnp.float32) * (-np.log(10000.0) / dim))
    pe = jnp.zeros((seq_len, dim), dtype=jnp.float32)
    pe = pe.at[:, 0::2].set(jnp.sin(pos * div))
    pe = pe.at[:, 1::2].set(jnp.cos(pos * div))
    return pe


def kernel(input_ids, table):
    batch, seq_len = input_ids.shape
    _, dim = table.shape
    assert batch == _NW * 128 and seq_len % 8 == 0
    assert dim % _LANES == 0 and dim % 8 == 0
    s_hi = seq_len // 8
    ids4 = (input_ids.astype(jnp.int32).T
            .reshape(s_hi, 8, _NW, 128).transpose(2, 0, 1, 3))
    pe = _pos_encoding(seq_len, dim)
    out5 = _make_emb_kernel(batch, seq_len, dim)(ids4, table, pe)
    return out5.transpose(2, 4, 0, 1, 3).reshape(batch, seq_len, dim)


# speedup vs baseline: 1.0405x; 1.0080x over previous
import functools

import jax
import jax.numpy as jnp
import numpy as np
from jax import lax
from jax.experimental import pallas as pl
from jax.experimental.pallas import tpu as pltpu
from jax.experimental.pallas import tpu_sc as plsc

_NUM_CORES = 2
_NUM_SUBCORES = 16
_LANES = 16
_NW = _NUM_CORES * _NUM_SUBCORES

_NBUF = 4
_AHEAD = 3
_BLK = 4


def _make_emb_kernel(batch, seq_len, dim):
    bpw = batch // _NW
    s_hi = seq_len // 8
    n_groups = seq_len // _NBUF
    cgroups = dim // _LANES
    mesh = plsc.VectorSubcoreMesh(
        core_axis_name="c", subcore_axis_name="s",
        num_cores=_NUM_CORES, num_subcores=_NUM_SUBCORES)

    @functools.partial(
        pl.kernel,
        out_type=jax.ShapeDtypeStruct(
            (seq_len, dim // 8, _NW, 8, 128), jnp.float32),
        mesh=mesh,
        scratch_types=[
            pltpu.VMEM((s_hi, 8, bpw), jnp.int32),
            pltpu.VMEM((_NBUF, bpw, dim), jnp.float32),
            pltpu.VMEM((_BLK, dim, bpw + 1), jnp.float32),
            pltpu.VMEM((seq_len, dim), jnp.float32),
            pltpu.SemaphoreType.DMA((_NBUF,)),
            pltpu.SemaphoreType.DMA((_BLK,)),
        ],
        compiler_params=pltpu.CompilerParams(
            use_tc_tiling_on_sc=False, needs_layout_passes=False),
    )
    def emb(ids_hbm, table_hbm, pe_hbm, out_hbm, idx_v, rows, block, pe_v,
            gsem, ssem):
        wid = lax.axis_index("c") * _NUM_SUBCORES + lax.axis_index("s")
        pltpu.sync_copy(pe_hbm, pe_v)
        pltpu.sync_copy(ids_hbm.at[wid], idx_v)
        iota = lax.iota(jnp.int32, _LANES)
        cvecs = [g * _LANES + iota for g in range(cgroups)]

        def gather_start(s, slot):
            pltpu.async_copy(
                table_hbm.at[idx_v.at[s // 8, lax.rem(s, 8)]],
                rows.at[slot], gsem.at[slot])

        def gather_wait(s, slot):
            pltpu.make_async_copy(
                table_hbm.at[idx_v.at[s // 8, lax.rem(s, 8)]],
                rows.at[slot], gsem.at[slot]).wait()

        def store_start(s, p):
            for chi in range(dim // 8):
                pltpu.async_copy(block.at[p, pl.ds(chi * 8, 8), pl.ds(0, 128)],
                                 out_hbm.at[s, chi, wid], ssem.at[p])

        def store_wait(s, p):
            for chi in range(dim // 8):
                pltpu.make_async_copy(
                    block.at[p, pl.ds(chi * 8, 8), pl.ds(0, 128)],
                    out_hbm.at[s, chi, wid], ssem.at[p]).wait()

        for b in range(_AHEAD):
            gather_start(b, b)

        def group_body(g, carry):
            for b in range(_NBUF):
                s = g * _NBUF + b
                p = b % _BLK
                gather_wait(s, b)

                @pl.when(s >= _BLK)
                def _drain():
                    store_wait(s - _BLK, p)

                pe_gs = [pe_v[s, pl.ds(gg * _LANES, _LANES)]
                         for gg in range(cgroups)]

                @plsc.parallel_loop(0, bpw, 1, unroll=16)
                def blo_body(blo):
                    blo_b = lax.broadcast(blo, (_LANES,))
                    for gg in range(cgroups):
                        val = rows[b, blo, pl.ds(gg * _LANES, _LANES)]
                        plsc.store_scatter(block.at[p], [cvecs[gg], blo_b],
                                           val + pe_gs[gg])

                f = s + _AHEAD

                @pl.when(f < seq_len)
                def _fire():
                    gather_start(f, (b + _AHEAD) % _NBUF)

                store_start(s, p)
            return carry

        lax.fori_loop(0, n_groups, group_body, 0)
        for i in range(_BLK):
            s_tail = seq_len - _BLK + i
            store_wait(s_tail, s_tail % _BLK)

    return emb


def _pos_encoding(seq_len, dim):
    pos = jnp.arange(seq_len, dtype=jnp.float32)[:, None]
    div = jnp.exp(
        jnp.arange(0, dim, 2, dtype=jnp.float32) * (-np.log(10000.0) / dim))
    pe = jnp.zeros((seq_len, dim), dtype=jnp.float32)
    pe = pe.at[:, 0::2].set(jnp.sin(pos * div))
    pe = pe.at[:, 1::2].set(jnp.cos(pos * div))
    return pe


def kernel(input_ids, table):
    batch, seq_len = input_ids.shape
    _, dim = table.shape
    assert batch == _NW * 128 and seq_len % 8 == 0
    assert dim % _LANES == 0 and dim % 8 == 0
    s_hi = seq_len // 8
    ids4 = (input_ids.astype(jnp.int32).T
            .reshape(s_hi, 8, _NW, 128).transpose(2, 0, 1, 3))
    pe = _pos_encoding(seq_len, dim)
    out5 = _make_emb_kernel(batch, seq_len, dim)(ids4, table, pe)
    return out5.transpose(2, 4, 0, 1, 3).reshape(batch, seq_len, dim)
